# baseline (speedup 1.0000x reference)
import functools

import jax
import jax.numpy as jnp
from jax import lax
from jax.experimental import pallas as pl
from jax.experimental.pallas import tpu as pltpu
from jax.experimental.pallas import tpu_sc as plsc

_NC = 2
_NS = 16
_NW = _NC * _NS


@functools.lru_cache(maxsize=None)
def _make_deg_kernel(N, E):
    e_per_w = E // _NW
    CH = 80
    n_ch = e_per_w // CH
    ZCH = 2000
    n_z = N // ZCH
    mesh = plsc.VectorSubcoreMesh(core_axis_name="c", subcore_axis_name="s")

    @functools.partial(
        pl.kernel,
        mesh=mesh,
        out_type=jax.ShapeDtypeStruct((_NC * N,), jnp.float32),
        scratch_types=[
            pltpu.VMEM((n_ch, CH), jnp.int32),
            pltpu.VMEM((CH,), jnp.float32),
            pltpu.VMEM((ZCH,), jnp.float32),
            pltpu.VMEM_SHARED((N,), jnp.float32),
            pltpu.SemaphoreType.DMA((8,)),
        ],
    )
    def deg_k(eidx_hbm, out_hbm, didx_v, ones_v, zbuf_v, acc_sh, sem):
        cid = lax.axis_index("c")
        sid = lax.axis_index("s")
        wid = sid * _NC + cid
        pltpu.sync_copy(eidx_hbm.at[1, wid], didx_v)

        def fill_ones(i, _):
            ones_v[pl.ds(i * 16, 16)] = jnp.ones((16,), jnp.float32)
            return 0

        lax.fori_loop(0, CH // 16, fill_ones, 0)

        def fill_zeros(i, _):
            zbuf_v[pl.ds(i * 16, 16)] = jnp.zeros((16,), jnp.float32)
            return 0

        lax.fori_loop(0, ZCH // 16, fill_zeros, 0)

        @pl.when(sid < n_z)
        def _():
            pltpu.sync_copy(zbuf_v, acc_sh.at[pl.ds(sid * ZCH, ZCH)])

        plsc.subcore_barrier()

        def body(i, _):
            r = lax.rem(i, 8)

            @pl.when(i >= 8)
            def _():
                pltpu.make_async_copy(ones_v, acc_sh.at[didx_v.at[i - 8]],
                                      sem.at[r]).wait()

            pltpu.async_copy(ones_v, acc_sh.at[didx_v.at[i]], sem.at[r],
                             add=True)
            return 0

        lax.fori_loop(0, n_ch, body, 0)

        def drain(i, _):
            r = lax.rem(i, 8)
            pltpu.make_async_copy(ones_v, acc_sh.at[didx_v.at[i]],
                                  sem.at[r]).wait()
            return 0

        lax.fori_loop(n_ch - 8, n_ch, drain, 0)
        plsc.subcore_barrier()

        @pl.when(sid < n_z)
        def _():
            pltpu.sync_copy(acc_sh.at[pl.ds(sid * ZCH, ZCH)], zbuf_v)
            pltpu.sync_copy(zbuf_v, out_hbm.at[pl.ds(cid * N + sid * ZCH, ZCH)])

    return deg_k


@functools.lru_cache(maxsize=None)
def _make_agg_kernel(N, E, F):
    e_per_w = E // _NW
    CH = 40
    NBUF = 5
    GRP = 10
    n_ch = e_per_w // CH
    n_grp = n_ch // GRP
    assert n_ch % NBUF == 0 and n_ch % GRP == 0 and GRP % NBUF == 0
    n_rz = N // CH
    n_rounds = (n_rz + _NS - 1) // _NS
    mesh = plsc.VectorSubcoreMesh(core_axis_name="c", subcore_axis_name="s")

    @functools.partial(
        pl.kernel,
        mesh=mesh,
        out_type=jax.ShapeDtypeStruct((_NC, N, F), jnp.float32),
        scratch_types=[
            pltpu.VMEM((e_per_w,), jnp.int32),
            pltpu.VMEM((2, GRP, CH), jnp.int32),
            pltpu.VMEM((NBUF, CH, F), jnp.float32),
            pltpu.VMEM_SHARED((N, F), jnp.float32),
            pltpu.SemaphoreType.DMA((NBUF,)),
            pltpu.SemaphoreType.DMA((2,)),
        ],
    )
    def agg_k(hs_hbm, eflat_hbm, eidx_hbm, out_hbm, sidx_v, dring_v, rows_v,
              acc_sh, sem, semd):
        cid = lax.axis_index("c")
        sid = lax.axis_index("s")
        wid = sid * _NC + cid

        pltpu.sync_copy(eflat_hbm.at[pl.ds(wid * e_per_w, e_per_w)], sidx_v)

        def fill_zeros(i, _):
            for j in range(F // 16):
                rows_v[0, i, pl.ds(j * 16, 16)] = jnp.zeros((16,),
                                                            jnp.float32)
            return 0

        lax.fori_loop(0, CH, fill_zeros, 0)

        def zloop(k, _):
            c = sid + k * _NS

            @pl.when(c < n_rz)
            def _():
                pltpu.sync_copy(rows_v.at[0], acc_sh.at[pl.ds(c * CH, CH)])

            return 0

        lax.fori_loop(0, n_rounds, zloop, 0)
        plsc.subcore_barrier()

        pltpu.async_copy(eidx_hbm.at[1, wid, 0], dring_v.at[0], semd.at[0])
        for b in range(NBUF):
            pltpu.async_copy(hs_hbm.at[sidx_v.at[pl.ds(b * CH, CH)]],
                             rows_v.at[b], sem.at[b])
        pltpu.make_async_copy(eidx_hbm.at[1, wid, 0], dring_v.at[0],
                              semd.at[0]).wait()

        def gbody(g, _):
            r = lax.rem(g, 2)

            @pl.when(g > 0)
            def _():
                pltpu.make_async_copy(eidx_hbm.at[1, wid, g],
                                      dring_v.at[r], semd.at[r]).wait()

            @pl.when(g + 1 < n_grp)
            def _():
                pltpu.async_copy(eidx_hbm.at[1, wid, g + 1],
                                 dring_v.at[1 - r], semd.at[1 - r])

            for k in range(GRP):
                b = k % NBUF
                i = g * GRP + k
                pltpu.make_async_copy(
                    hs_hbm.at[sidx_v.at[pl.ds(i * CH, CH)]],
                    rows_v.at[b], sem.at[b]
                ).wait()
                pltpu.sync_copy(rows_v.at[b], acc_sh.at[dring_v.at[r, k]],
                                add=True)
                j = i + NBUF

                @pl.when(j < n_ch)
                def _():
                    pltpu.async_copy(
                        hs_hbm.at[sidx_v.at[pl.ds(j * CH, CH)]],
                        rows_v.at[b], sem.at[b])

            return 0

        lax.fori_loop(0, n_grp, gbody, 0)
        plsc.subcore_barrier()

        def cpout(k, _):
            c = sid + k * _NS

            @pl.when(c < n_rz)
            def _():
                pltpu.sync_copy(acc_sh.at[pl.ds(c * CH, CH)], rows_v.at[0])
                pltpu.sync_copy(rows_v.at[0], out_hbm.at[cid, pl.ds(c * CH, CH)])

            return 0

        lax.fori_loop(0, n_rounds, cpout, 0)

    return agg_k


_BLK = 2000


def _tc_matmul(x, W1):
    N, IN = x.shape
    H = W1.shape[1]

    def body(x_ref, w_ref, h_ref):
        h_ref[...] = jnp.dot(x_ref[...], w_ref[...],
                             preferred_element_type=jnp.float32)

    return pl.pallas_call(
        body,
        grid=(N // _BLK,),
        in_specs=[
            pl.BlockSpec((_BLK, IN), lambda i: (i, 0)),
            pl.BlockSpec((IN, H), lambda i: (0, 0)),
        ],
        out_specs=pl.BlockSpec((_BLK, H), lambda i: (i, 0)),
        out_shape=jax.ShapeDtypeStruct((N, H), jnp.float32),
    )(x, W1)


def _tc_scale(degT, h1):
    N, H = h1.shape

    def body(deg_ref, h_ref, hs_ref, dinv_ref):
        d = deg_ref[:, 0] + deg_ref[:, 1] + 1.0
        dinv = lax.rsqrt(d)
        hs_ref[...] = h_ref[...] * dinv[:, None]
        dinv_ref[...] = dinv[:, None]

    return pl.pallas_call(
        body,
        grid=(N // _BLK,),
        in_specs=[
            pl.BlockSpec((_BLK, 2), lambda i: (i, 0)),
            pl.BlockSpec((_BLK, H), lambda i: (i, 0)),
        ],
        out_specs=[
            pl.BlockSpec((_BLK, H), lambda i: (i, 0)),
            pl.BlockSpec((_BLK, 1), lambda i: (i, 0)),
        ],
        out_shape=[
            jax.ShapeDtypeStruct((N, H), jnp.float32),
            jax.ShapeDtypeStruct((N, 1), jnp.float32),
        ],
    )(degT, h1)


def _tc_mid(dinv, hs1, agg1, W2, b1):
    N, H = hs1.shape

    def body(dinv_ref, hs_ref, agg_ref, w_ref, b_ref, out_ref):
        dinv = dinv_ref[...]
        a = agg_ref[0] + agg_ref[1] + hs_ref[...]
        t = jnp.maximum(a * dinv + b_ref[...], 0.0)
        out_ref[...] = (
            jnp.dot(t, w_ref[...], preferred_element_type=jnp.float32) * dinv
        )

    return pl.pallas_call(
        body,
        grid=(N // _BLK,),
        in_specs=[
            pl.BlockSpec((_BLK, 1), lambda i: (i, 0)),
            pl.BlockSpec((_BLK, H), lambda i: (i, 0)),
            pl.BlockSpec((2, _BLK, H), lambda i: (0, i, 0)),
            pl.BlockSpec((H, H), lambda i: (0, 0)),
            pl.BlockSpec((1, H), lambda i: (0, 0)),
        ],
        out_specs=pl.BlockSpec((_BLK, H), lambda i: (i, 0)),
        out_shape=jax.ShapeDtypeStruct((N, H), jnp.float32),
    )(dinv, hs1, agg1, W2, b1)


def _tc_last(dinv, hs2, agg2, b2, batchT, Wfc, bfc, G):
    N, H = hs2.shape
    OUT = Wfc.shape[1]
    nsteps = N // _BLK

    def body(dinv_ref, hs_ref, agg_ref, b_ref, batch_ref, wfc_ref, bfc_ref,
             y_ref, u_ref):
        i = pl.program_id(0)
        dinv = dinv_ref[...]
        z = (agg_ref[0] + agg_ref[1] + hs_ref[...]) * dinv + b_ref[...]
        cols = lax.broadcasted_iota(jnp.int32, (_BLK, G), 1)
        oh = (batch_ref[...] == cols).astype(jnp.float32)
        pu = lax.dot_general(
            oh, z, (((0,), (0,)), ((), ())),
            preferred_element_type=jnp.float32,
        )

        @pl.when(i == 0)
        def _():
            u_ref[...] = jnp.zeros_like(u_ref)

        u_ref[...] += pu

        @pl.when(i == nsteps - 1)
        def _():
            y_ref[...] = (
                jnp.dot(u_ref[...], wfc_ref[...],
                        preferred_element_type=jnp.float32)
                + bfc_ref[...]
            )

    return pl.pallas_call(
        body,
        grid=(nsteps,),
        in_specs=[
            pl.BlockSpec((_BLK, 1), lambda i: (i, 0)),
            pl.BlockSpec((_BLK, H), lambda i: (i, 0)),
            pl.BlockSpec((2, _BLK, H), lambda i: (0, i, 0)),
            pl.BlockSpec((1, H), lambda i: (0, 0)),
            pl.BlockSpec((_BLK, 1), lambda i: (i, 0)),
            pl.BlockSpec((H, OUT), lambda i: (0, 0)),
            pl.BlockSpec((1, OUT), lambda i: (0, 0)),
        ],
        out_specs=[
            pl.BlockSpec((G, OUT), lambda i: (0, 0)),
            pl.BlockSpec((G, H), lambda i: (0, 0)),
        ],
        out_shape=[
            jax.ShapeDtypeStruct((G, OUT), jnp.float32),
            jax.ShapeDtypeStruct((G, H), jnp.float32),
        ],
    )(dinv, hs2, agg2, b2, batchT, Wfc, bfc)


def kernel(x, edge_index, batch, W1, b1, W2, b2, Wfc, bfc):
    N, IN = x.shape
    H = W1.shape[1]
    OUT = Wfc.shape[1]
    E = edge_index.shape[1]
    G = 64

    eflat = edge_index.reshape(2 * E)
    eidx4 = edge_index.reshape(2, _NW, -1, 80)
    eidx5 = edge_index.reshape(2, _NW, -1, 10, 40)

    degp = _make_deg_kernel(N, E)(eidx4)
    h1 = _tc_matmul(x, W1)
    degT = jnp.transpose(degp.reshape(_NC, N))

    hs1, dinv = _tc_scale(degT, h1)
    agg1 = _make_agg_kernel(N, E, H)(hs1, eflat, eidx5)
    hs2 = _tc_mid(dinv, hs1, agg1, W2, b1.reshape(1, H))
    agg2 = _make_agg_kernel(N, E, H)(hs2, eflat, eidx5)
    y, u = _tc_last(dinv, hs2, agg2, b2.reshape(1, H),
                    batch.reshape(N, 1), Wfc, bfc.reshape(1, OUT), G)
    return (y, u)


# speedup vs baseline: 1.0300x; 1.0300x over previous
import functools

import jax
import jax.numpy as jnp
from jax import lax
from jax.experimental import pallas as pl
from jax.experimental.pallas import tpu as pltpu
from jax.experimental.pallas import tpu_sc as plsc

_NC = 2
_NS = 16
_NW = _NC * _NS


@functools.lru_cache(maxsize=None)
def _make_deg_kernel(N, E):
    e_per_w = E // _NW
    CH = 80
    n_ch = e_per_w // CH
    ZCH = 2000
    n_z = N // ZCH
    mesh = plsc.VectorSubcoreMesh(core_axis_name="c", subcore_axis_name="s")

    @functools.partial(
        pl.kernel,
        mesh=mesh,
        out_type=jax.ShapeDtypeStruct((_NC * N,), jnp.float32),
        scratch_types=[
            pltpu.VMEM((n_ch, CH), jnp.int32),
            pltpu.VMEM((CH,), jnp.float32),
            pltpu.VMEM((ZCH,), jnp.float32),
            pltpu.VMEM_SHARED((N,), jnp.float32),
            pltpu.SemaphoreType.DMA((8,)),
        ],
    )
    def deg_k(eidx_hbm, out_hbm, didx_v, ones_v, zbuf_v, acc_sh, sem):
        cid = lax.axis_index("c")
        sid = lax.axis_index("s")
        wid = sid * _NC + cid
        pltpu.sync_copy(eidx_hbm.at[1, wid], didx_v)

        def fill_ones(i, _):
            ones_v[pl.ds(i * 16, 16)] = jnp.ones((16,), jnp.float32)
            return 0

        lax.fori_loop(0, CH // 16, fill_ones, 0)

        def fill_zeros(i, _):
            zbuf_v[pl.ds(i * 16, 16)] = jnp.zeros((16,), jnp.float32)
            return 0

        lax.fori_loop(0, ZCH // 16, fill_zeros, 0)

        @pl.when(sid < n_z)
        def _():
            pltpu.sync_copy(zbuf_v, acc_sh.at[pl.ds(sid * ZCH, ZCH)])

        plsc.subcore_barrier()

        def body(i, _):
            r = lax.rem(i, 8)

            @pl.when(i >= 8)
            def _():
                pltpu.make_async_copy(ones_v, acc_sh.at[didx_v.at[i - 8]],
                                      sem.at[r]).wait()

            pltpu.async_copy(ones_v, acc_sh.at[didx_v.at[i]], sem.at[r],
                             add=True)
            return 0

        lax.fori_loop(0, n_ch, body, 0)

        def drain(i, _):
            r = lax.rem(i, 8)
            pltpu.make_async_copy(ones_v, acc_sh.at[didx_v.at[i]],
                                  sem.at[r]).wait()
            return 0

        lax.fori_loop(n_ch - 8, n_ch, drain, 0)
        plsc.subcore_barrier()

        @pl.when(sid < n_z)
        def _():
            pltpu.sync_copy(acc_sh.at[pl.ds(sid * ZCH, ZCH)], zbuf_v)
            pltpu.sync_copy(zbuf_v, out_hbm.at[pl.ds(cid * N + sid * ZCH, ZCH)])

    return deg_k


@functools.lru_cache(maxsize=None)
def _make_agg_kernel(N, E, F):
    e_per_w = E // _NW
    CH = 40
    NBUF = 5
    GRP = 10
    n_ch = e_per_w // CH
    n_grp = n_ch // GRP
    assert n_ch % NBUF == 0 and n_ch % GRP == 0 and GRP % NBUF == 0
    n_rz = N // CH
    n_rounds = (n_rz + _NS - 1) // _NS
    mesh = plsc.VectorSubcoreMesh(core_axis_name="c", subcore_axis_name="s")

    @functools.partial(
        pl.kernel,
        mesh=mesh,
        out_type=jax.ShapeDtypeStruct((_NC, N, F), jnp.float32),
        scratch_types=[
            pltpu.VMEM((e_per_w,), jnp.int32),
            pltpu.VMEM((2, GRP, CH), jnp.int32),
            pltpu.VMEM((NBUF, CH, F), jnp.float32),
            pltpu.VMEM_SHARED((N, F), jnp.float32),
            pltpu.SemaphoreType.DMA((NBUF,)),
            pltpu.SemaphoreType.DMA((2,)),
        ],
    )
    def agg_k(hs_hbm, eflat_hbm, eidx_hbm, out_hbm, sidx_v, dring_v, rows_v,
              acc_sh, sem, semd):
        cid = lax.axis_index("c")
        sid = lax.axis_index("s")
        wid = sid * _NC + cid

        pltpu.sync_copy(eflat_hbm.at[pl.ds(wid * e_per_w, e_per_w)], sidx_v)

        def fill_zeros(i, _):
            for j in range(F // 16):
                rows_v[0, i, pl.ds(j * 16, 16)] = jnp.zeros((16,),
                                                            jnp.float32)
            return 0

        lax.fori_loop(0, CH, fill_zeros, 0)

        def zloop(k, _):
            c = sid + k * _NS

            @pl.when(c < n_rz)
            def _():
                pltpu.sync_copy(rows_v.at[0], acc_sh.at[pl.ds(c * CH, CH)])

            return 0

        lax.fori_loop(0, n_rounds, zloop, 0)
        plsc.subcore_barrier()

        pltpu.async_copy(eidx_hbm.at[1, wid, 0], dring_v.at[0], semd.at[0])
        for b in range(NBUF):
            pltpu.async_copy(hs_hbm.at[sidx_v.at[pl.ds(b * CH, CH)]],
                             rows_v.at[b], sem.at[b])
        pltpu.make_async_copy(eidx_hbm.at[1, wid, 0], dring_v.at[0],
                              semd.at[0]).wait()

        def gbody(g, _):
            r = lax.rem(g, 2)

            @pl.when(g > 0)
            def _():
                pltpu.make_async_copy(eidx_hbm.at[1, wid, g],
                                      dring_v.at[r], semd.at[r]).wait()

            @pl.when(g + 1 < n_grp)
            def _():
                pltpu.async_copy(eidx_hbm.at[1, wid, g + 1],
                                 dring_v.at[1 - r], semd.at[1 - r])

            for k in range(GRP):
                b = k % NBUF
                i = g * GRP + k
                pltpu.make_async_copy(
                    hs_hbm.at[sidx_v.at[pl.ds(i * CH, CH)]],
                    rows_v.at[b], sem.at[b]
                ).wait()
                pltpu.sync_copy(rows_v.at[b], acc_sh.at[dring_v.at[r, k]],
                                add=True)
                j = i + NBUF

                @pl.when(j < n_ch)
                def _():
                    pltpu.async_copy(
                        hs_hbm.at[sidx_v.at[pl.ds(j * CH, CH)]],
                        rows_v.at[b], sem.at[b])

            return 0

        lax.fori_loop(0, n_grp, gbody, 0)
        plsc.subcore_barrier()

        def cpout(k, _):
            c = sid + k * _NS

            @pl.when(c < n_rz)
            def _():
                pltpu.sync_copy(acc_sh.at[pl.ds(c * CH, CH)], rows_v.at[0])
                pltpu.sync_copy(rows_v.at[0], out_hbm.at[cid, pl.ds(c * CH, CH)])

            return 0

        lax.fori_loop(0, n_rounds, cpout, 0)

    return agg_k


_BLK = 2000


def _tc_first(degT, x, W1):
    N, IN = x.shape
    H = W1.shape[1]

    def body(deg_ref, x_ref, w_ref, hs_ref, dinv_ref):
        d = deg_ref[:, 0] + deg_ref[:, 1] + 1.0
        dinv = lax.rsqrt(d)
        h = jnp.dot(x_ref[...], w_ref[...], preferred_element_type=jnp.float32)
        hs_ref[...] = h * dinv[:, None]
        dinv_ref[...] = dinv[:, None]

    return pl.pallas_call(
        body,
        grid=(N // _BLK,),
        in_specs=[
            pl.BlockSpec((_BLK, 2), lambda i: (i, 0)),
            pl.BlockSpec((_BLK, IN), lambda i: (i, 0)),
            pl.BlockSpec((IN, H), lambda i: (0, 0)),
        ],
        out_specs=[
            pl.BlockSpec((_BLK, H), lambda i: (i, 0)),
            pl.BlockSpec((_BLK, 1), lambda i: (i, 0)),
        ],
        out_shape=[
            jax.ShapeDtypeStruct((N, H), jnp.float32),
            jax.ShapeDtypeStruct((N, 1), jnp.float32),
        ],
    )(degT, x, W1)


def _tc_mid(dinv, hs1, agg1, W2, b1):
    N, H = hs1.shape

    def body(dinv_ref, hs_ref, agg_ref, w_ref, b_ref, out_ref):
        dinv = dinv_ref[...]
        a = agg_ref[0] + agg_ref[1] + hs_ref[...]
        t = jnp.maximum(a * dinv + b_ref[...], 0.0)
        out_ref[...] = (
            jnp.dot(t, w_ref[...], preferred_element_type=jnp.float32) * dinv
        )

    return pl.pallas_call(
        body,
        grid=(N // _BLK,),
        in_specs=[
            pl.BlockSpec((_BLK, 1), lambda i: (i, 0)),
            pl.BlockSpec((_BLK, H), lambda i: (i, 0)),
            pl.BlockSpec((2, _BLK, H), lambda i: (0, i, 0)),
            pl.BlockSpec((H, H), lambda i: (0, 0)),
            pl.BlockSpec((1, H), lambda i: (0, 0)),
        ],
        out_specs=pl.BlockSpec((_BLK, H), lambda i: (i, 0)),
        out_shape=jax.ShapeDtypeStruct((N, H), jnp.float32),
    )(dinv, hs1, agg1, W2, b1)


def _tc_last(dinv, hs2, agg2, b2, batchT, Wfc, bfc, G):
    N, H = hs2.shape
    OUT = Wfc.shape[1]
    nsteps = N // _BLK

    def body(dinv_ref, hs_ref, agg_ref, b_ref, batch_ref, wfc_ref, bfc_ref,
             y_ref, u_ref):
        i = pl.program_id(0)
        dinv = dinv_ref[...]
        z = (agg_ref[0] + agg_ref[1] + hs_ref[...]) * dinv + b_ref[...]
        cols = lax.broadcasted_iota(jnp.int32, (_BLK, G), 1)
        oh = (batch_ref[...] == cols).astype(jnp.float32)
        pu = lax.dot_general(
            oh, z, (((0,), (0,)), ((), ())),
            preferred_element_type=jnp.float32,
        )

        @pl.when(i == 0)
        def _():
            u_ref[...] = jnp.zeros_like(u_ref)

        u_ref[...] += pu

        @pl.when(i == nsteps - 1)
        def _():
            y_ref[...] = (
                jnp.dot(u_ref[...], wfc_ref[...],
                        preferred_element_type=jnp.float32)
                + bfc_ref[...]
            )

    return pl.pallas_call(
        body,
        grid=(nsteps,),
        in_specs=[
            pl.BlockSpec((_BLK, 1), lambda i: (i, 0)),
            pl.BlockSpec((_BLK, H), lambda i: (i, 0)),
            pl.BlockSpec((2, _BLK, H), lambda i: (0, i, 0)),
            pl.BlockSpec((1, H), lambda i: (0, 0)),
            pl.BlockSpec((_BLK, 1), lambda i: (i, 0)),
            pl.BlockSpec((H, OUT), lambda i: (0, 0)),
            pl.BlockSpec((1, OUT), lambda i: (0, 0)),
        ],
        out_specs=[
            pl.BlockSpec((G, OUT), lambda i: (0, 0)),
            pl.BlockSpec((G, H), lambda i: (0, 0)),
        ],
        out_shape=[
            jax.ShapeDtypeStruct((G, OUT), jnp.float32),
            jax.ShapeDtypeStruct((G, H), jnp.float32),
        ],
    )(dinv, hs2, agg2, b2, batchT, Wfc, bfc)


def kernel(x, edge_index, batch, W1, b1, W2, b2, Wfc, bfc):
    N, IN = x.shape
    H = W1.shape[1]
    OUT = Wfc.shape[1]
    E = edge_index.shape[1]
    G = 64

    eflat = edge_index.reshape(2 * E)
    eidx4 = edge_index.reshape(2, _NW, -1, 80)
    eidx5 = edge_index.reshape(2, _NW, -1, 10, 40)

    degp = _make_deg_kernel(N, E)(eidx4)
    degT = jnp.transpose(degp.reshape(_NC, N))

    hs1, dinv = _tc_first(degT, x, W1)
    agg1 = _make_agg_kernel(N, E, H)(hs1, eflat, eidx5)
    hs2 = _tc_mid(dinv, hs1, agg1, W2, b1.reshape(1, H))
    agg2 = _make_agg_kernel(N, E, H)(hs2, eflat, eidx5)
    y, u = _tc_last(dinv, hs2, agg2, b2.reshape(1, H),
                    batch.reshape(N, 1), Wfc, bfc.reshape(1, OUT), G)
    return (y, u)


# speedup vs baseline: 1.0560x; 1.0253x over previous
import functools

import jax
import jax.numpy as jnp
from jax import lax
from jax.experimental import pallas as pl
from jax.experimental.pallas import tpu as pltpu
from jax.experimental.pallas import tpu_sc as plsc

_NC = 2
_NS = 16
_NW = _NC * _NS


@functools.lru_cache(maxsize=None)
def _make_deg_kernel(N, E):
    e_per_w = E // _NW
    CH = 80
    n_ch = e_per_w // CH
    ZCH = 2000
    n_z = N // ZCH
    mesh = plsc.VectorSubcoreMesh(core_axis_name="c", subcore_axis_name="s")

    @functools.partial(
        pl.kernel,
        mesh=mesh,
        out_type=jax.ShapeDtypeStruct((_NC * N,), jnp.float32),
        scratch_types=[
            pltpu.VMEM((n_ch, CH), jnp.int32),
            pltpu.VMEM((CH,), jnp.float32),
            pltpu.VMEM((ZCH,), jnp.float32),
            pltpu.VMEM_SHARED((N,), jnp.float32),
            pltpu.SemaphoreType.DMA((8,)),
        ],
    )
    def deg_k(eidx_hbm, out_hbm, didx_v, ones_v, zbuf_v, acc_sh, sem):
        cid = lax.axis_index("c")
        sid = lax.axis_index("s")
        wid = sid * _NC + cid
        pltpu.sync_copy(eidx_hbm.at[1, wid], didx_v)

        def fill_ones(i, _):
            ones_v[pl.ds(i * 16, 16)] = jnp.ones((16,), jnp.float32)
            return 0

        lax.fori_loop(0, CH // 16, fill_ones, 0)

        def fill_zeros(i, _):
            zbuf_v[pl.ds(i * 16, 16)] = jnp.zeros((16,), jnp.float32)
            return 0

        lax.fori_loop(0, ZCH // 16, fill_zeros, 0)

        @pl.when(sid < n_z)
        def _():
            pltpu.sync_copy(zbuf_v, acc_sh.at[pl.ds(sid * ZCH, ZCH)])

        plsc.subcore_barrier()

        def body(i, _):
            r = lax.rem(i, 8)

            @pl.when(i >= 8)
            def _():
                pltpu.make_async_copy(ones_v, acc_sh.at[didx_v.at[i - 8]],
                                      sem.at[r]).wait()

            pltpu.async_copy(ones_v, acc_sh.at[didx_v.at[i]], sem.at[r],
                             add=True)
            return 0

        lax.fori_loop(0, n_ch, body, 0)

        def drain(i, _):
            r = lax.rem(i, 8)
            pltpu.make_async_copy(ones_v, acc_sh.at[didx_v.at[i]],
                                  sem.at[r]).wait()
            return 0

        lax.fori_loop(n_ch - 8, n_ch, drain, 0)
        plsc.subcore_barrier()

        @pl.when(sid < n_z)
        def _():
            pltpu.sync_copy(acc_sh.at[pl.ds(sid * ZCH, ZCH)], zbuf_v)
            pltpu.sync_copy(zbuf_v, out_hbm.at[pl.ds(cid * N + sid * ZCH, ZCH)])

    return deg_k


@functools.lru_cache(maxsize=None)
def _make_agg_kernel(N, E, F):
    e_per_w = E // _NW
    CH = 40
    NBUF = 5
    GRP = 10
    n_ch = e_per_w // CH
    n_grp = n_ch // GRP
    assert n_ch % NBUF == 0 and n_ch % GRP == 0 and GRP % NBUF == 0
    n_rz = N // CH
    n_rounds = (n_rz + _NS - 1) // _NS
    mesh = plsc.VectorSubcoreMesh(core_axis_name="c", subcore_axis_name="s")

    @functools.partial(
        pl.kernel,
        mesh=mesh,
        out_type=jax.ShapeDtypeStruct((_NC, N, F), jnp.float32),
        scratch_types=[
            pltpu.VMEM((e_per_w,), jnp.int32),
            pltpu.VMEM((2, GRP, CH), jnp.int32),
            pltpu.VMEM((NBUF, CH, F), jnp.float32),
            pltpu.VMEM_SHARED((N, F), jnp.float32),
            pltpu.SemaphoreType.DMA((NBUF,)),
            pltpu.SemaphoreType.DMA((2,)),
        ],
    )
    def agg_k(hs_hbm, eflat_hbm, eidx_hbm, out_hbm, sidx_v, dring_v, rows_v,
              acc_sh, sem, semd):
        cid = lax.axis_index("c")
        sid = lax.axis_index("s")
        wid = sid * _NC + cid

        pltpu.sync_copy(eflat_hbm.at[pl.ds(wid * e_per_w, e_per_w)], sidx_v)

        def fill_zeros(i, _):
            for j in range(F // 16):
                rows_v[0, i, pl.ds(j * 16, 16)] = jnp.zeros((16,),
                                                            jnp.float32)
            return 0

        lax.fori_loop(0, CH, fill_zeros, 0)

        def zloop(k, _):
            c = sid + k * _NS

            @pl.when(c < n_rz)
            def _():
                pltpu.sync_copy(rows_v.at[0], acc_sh.at[pl.ds(c * CH, CH)])

            return 0

        lax.fori_loop(0, n_rounds, zloop, 0)
        plsc.subcore_barrier()

        pltpu.async_copy(eidx_hbm.at[1, wid, 0], dring_v.at[0], semd.at[0])
        for b in range(NBUF):
            pltpu.async_copy(hs_hbm.at[sidx_v.at[pl.ds(b * CH, CH)]],
                             rows_v.at[b], sem.at[b])
        pltpu.make_async_copy(eidx_hbm.at[1, wid, 0], dring_v.at[0],
                              semd.at[0]).wait()

        def gbody(g, _):
            r = lax.rem(g, 2)

            @pl.when(g > 0)
            def _():
                pltpu.make_async_copy(eidx_hbm.at[1, wid, g],
                                      dring_v.at[r], semd.at[r]).wait()

            @pl.when(g + 1 < n_grp)
            def _():
                pltpu.async_copy(eidx_hbm.at[1, wid, g + 1],
                                 dring_v.at[1 - r], semd.at[1 - r])

            for k in range(GRP):
                b = k % NBUF
                i = g * GRP + k
                pltpu.make_async_copy(
                    hs_hbm.at[sidx_v.at[pl.ds(i * CH, CH)]],
                    rows_v.at[b], sem.at[b]
                ).wait()
                pltpu.sync_copy(rows_v.at[b], acc_sh.at[dring_v.at[r, k]],
                                add=True)
                j = i + NBUF

                @pl.when(j < n_ch)
                def _():
                    pltpu.async_copy(
                        hs_hbm.at[sidx_v.at[pl.ds(j * CH, CH)]],
                        rows_v.at[b], sem.at[b])

            return 0

        lax.fori_loop(0, n_grp, gbody, 0)
        plsc.subcore_barrier()

        def cpout(k, _):
            c = sid + k * _NS
            r = lax.rem(k, 2)

            @pl.when(c < n_rz)
            def _():
                @pl.when(k >= 2)
                def _():
                    pltpu.make_async_copy(
                        rows_v.at[r],
                        out_hbm.at[cid, pl.ds((c - 2 * _NS) * CH, CH)],
                        sem.at[r]).wait()

                pltpu.sync_copy(acc_sh.at[pl.ds(c * CH, CH)], rows_v.at[r])
                pltpu.async_copy(rows_v.at[r],
                                 out_hbm.at[cid, pl.ds(c * CH, CH)],
                                 sem.at[r])

            return 0

        lax.fori_loop(0, n_rounds, cpout, 0)

        def cdrain(k, _):
            c = sid + k * _NS
            r = lax.rem(k, 2)

            @pl.when(c < n_rz)
            def _():
                pltpu.make_async_copy(rows_v.at[r],
                                      out_hbm.at[cid, pl.ds(c * CH, CH)],
                                      sem.at[r]).wait()

            return 0

        lax.fori_loop(n_rounds - 2, n_rounds, cdrain, 0)

    return agg_k


_BLK = 2000


def _tc_first(degT, x, W1):
    N, IN = x.shape
    H = W1.shape[1]

    def body(deg_ref, x_ref, w_ref, hs_ref, dinv_ref):
        d = deg_ref[:, 0] + deg_ref[:, 1] + 1.0
        dinv = lax.rsqrt(d)
        h = jnp.dot(x_ref[...], w_ref[...], preferred_element_type=jnp.float32)
        hs_ref[...] = h * dinv[:, None]
        dinv_ref[...] = dinv[:, None]

    return pl.pallas_call(
        body,
        grid=(N // _BLK,),
        in_specs=[
            pl.BlockSpec((_BLK, 2), lambda i: (i, 0)),
            pl.BlockSpec((_BLK, IN), lambda i: (i, 0)),
            pl.BlockSpec((IN, H), lambda i: (0, 0)),
        ],
        out_specs=[
            pl.BlockSpec((_BLK, H), lambda i: (i, 0)),
            pl.BlockSpec((_BLK, 1), lambda i: (i, 0)),
        ],
        out_shape=[
            jax.ShapeDtypeStruct((N, H), jnp.float32),
            jax.ShapeDtypeStruct((N, 1), jnp.float32),
        ],
    )(degT, x, W1)


def _tc_mid(dinv, hs1, agg1, W2, b1):
    N, H = hs1.shape

    def body(dinv_ref, hs_ref, agg_ref, w_ref, b_ref, out_ref):
        dinv = dinv_ref[...]
        a = agg_ref[0] + agg_ref[1] + hs_ref[...]
        t = jnp.maximum(a * dinv + b_ref[...], 0.0)
        out_ref[...] = (
            jnp.dot(t, w_ref[...], preferred_element_type=jnp.float32) * dinv
        )

    return pl.pallas_call(
        body,
        grid=(N // _BLK,),
        in_specs=[
            pl.BlockSpec((_BLK, 1), lambda i: (i, 0)),
            pl.BlockSpec((_BLK, H), lambda i: (i, 0)),
            pl.BlockSpec((2, _BLK, H), lambda i: (0, i, 0)),
            pl.BlockSpec((H, H), lambda i: (0, 0)),
            pl.BlockSpec((1, H), lambda i: (0, 0)),
        ],
        out_specs=pl.BlockSpec((_BLK, H), lambda i: (i, 0)),
        out_shape=jax.ShapeDtypeStruct((N, H), jnp.float32),
    )(dinv, hs1, agg1, W2, b1)


def _tc_last(dinv, hs2, agg2, b2, batchT, Wfc, bfc, G):
    N, H = hs2.shape
    OUT = Wfc.shape[1]
    nsteps = N // _BLK

    def body(dinv_ref, hs_ref, agg_ref, b_ref, batch_ref, wfc_ref, bfc_ref,
             y_ref, u_ref):
        i = pl.program_id(0)
        dinv = dinv_ref[...]
        z = (agg_ref[0] + agg_ref[1] + hs_ref[...]) * dinv + b_ref[...]
        cols = lax.broadcasted_iota(jnp.int32, (_BLK, G), 1)
        oh = (batch_ref[...] == cols).astype(jnp.float32)
        pu = lax.dot_general(
            oh, z, (((0,), (0,)), ((), ())),
            preferred_element_type=jnp.float32,
        )

        @pl.when(i == 0)
        def _():
            u_ref[...] = jnp.zeros_like(u_ref)

        u_ref[...] += pu

        @pl.when(i == nsteps - 1)
        def _():
            y_ref[...] = (
                jnp.dot(u_ref[...], wfc_ref[...],
                        preferred_element_type=jnp.float32)
                + bfc_ref[...]
            )

    return pl.pallas_call(
        body,
        grid=(nsteps,),
        in_specs=[
            pl.BlockSpec((_BLK, 1), lambda i: (i, 0)),
            pl.BlockSpec((_BLK, H), lambda i: (i, 0)),
            pl.BlockSpec((2, _BLK, H), lambda i: (0, i, 0)),
            pl.BlockSpec((1, H), lambda i: (0, 0)),
            pl.BlockSpec((_BLK, 1), lambda i: (i, 0)),
            pl.BlockSpec((H, OUT), lambda i: (0, 0)),
            pl.BlockSpec((1, OUT), lambda i: (0, 0)),
        ],
        out_specs=[
            pl.BlockSpec((G, OUT), lambda i: (0, 0)),
            pl.BlockSpec((G, H), lambda i: (0, 0)),
        ],
        out_shape=[
            jax.ShapeDtypeStruct((G, OUT), jnp.float32),
            jax.ShapeDtypeStruct((G, H), jnp.float32),
        ],
    )(dinv, hs2, agg2, b2, batchT, Wfc, bfc)


def kernel(x, edge_index, batch, W1, b1, W2, b2, Wfc, bfc):
    N, IN = x.shape
    H = W1.shape[1]
    OUT = Wfc.shape[1]
    E = edge_index.shape[1]
    G = 64

    eflat = edge_index.reshape(2 * E)
    eidx4 = edge_index.reshape(2, _NW, -1, 80)
    eidx5 = edge_index.reshape(2, _NW, -1, 10, 40)

    degp = _make_deg_kernel(N, E)(eidx4)
    degT = jnp.transpose(degp.reshape(_NC, N))

    hs1, dinv = _tc_first(degT, x, W1)
    agg1 = _make_agg_kernel(N, E, H)(hs1, eflat, eidx5)
    hs2 = _tc_mid(dinv, hs1, agg1, W2, b1.reshape(1, H))
    agg2 = _make_agg_kernel(N, E, H)(hs2, eflat, eidx5)
    y, u = _tc_last(dinv, hs2, agg2, b2.reshape(1, H),
                    batch.reshape(N, 1), Wfc, bfc.reshape(1, OUT), G)
    return (y, u)


# speedup vs baseline: 1.0664x; 1.0098x over previous
import functools

import jax
import jax.numpy as jnp
from jax import lax
from jax.experimental import pallas as pl
from jax.experimental.pallas import tpu as pltpu
from jax.experimental.pallas import tpu_sc as plsc

_NC = 2
_NS = 16
_NW = _NC * _NS


@functools.lru_cache(maxsize=None)
def _make_deg_kernel(N, E):
    e_per_w = E // _NW
    CH = 80
    n_ch = e_per_w // CH
    ZCH = 2000
    n_z = N // ZCH
    mesh = plsc.VectorSubcoreMesh(core_axis_name="c", subcore_axis_name="s")

    @functools.partial(
        pl.kernel,
        mesh=mesh,
        out_type=jax.ShapeDtypeStruct((_NC * N,), jnp.float32),
        scratch_types=[
            pltpu.VMEM((n_ch, CH), jnp.int32),
            pltpu.VMEM((CH,), jnp.float32),
            pltpu.VMEM((ZCH,), jnp.float32),
            pltpu.VMEM_SHARED((N,), jnp.float32),
            pltpu.SemaphoreType.DMA((8,)),
        ],
    )
    def deg_k(eidx_hbm, out_hbm, didx_v, ones_v, zbuf_v, acc_sh, sem):
        cid = lax.axis_index("c")
        sid = lax.axis_index("s")
        wid = sid * _NC + cid
        pltpu.async_copy(eidx_hbm.at[1, wid], didx_v, sem.at[0])

        def fill_ones(i, _):
            ones_v[pl.ds(i * 16, 16)] = jnp.ones((16,), jnp.float32)
            return 0

        lax.fori_loop(0, CH // 16, fill_ones, 0)

        def fill_zeros(i, _):
            zbuf_v[pl.ds(i * 16, 16)] = jnp.zeros((16,), jnp.float32)
            return 0

        lax.fori_loop(0, ZCH // 16, fill_zeros, 0)

        @pl.when(sid < n_z)
        def _():
            pltpu.sync_copy(zbuf_v, acc_sh.at[pl.ds(sid * ZCH, ZCH)])

        pltpu.make_async_copy(eidx_hbm.at[1, wid], didx_v, sem.at[0]).wait()
        plsc.subcore_barrier()

        def body(i, _):
            r = lax.rem(i, 8)

            @pl.when(i >= 8)
            def _():
                pltpu.make_async_copy(ones_v, acc_sh.at[didx_v.at[i - 8]],
                                      sem.at[r]).wait()

            pltpu.async_copy(ones_v, acc_sh.at[didx_v.at[i]], sem.at[r],
                             add=True)
            return 0

        lax.fori_loop(0, n_ch, body, 0)

        def drain(i, _):
            r = lax.rem(i, 8)
            pltpu.make_async_copy(ones_v, acc_sh.at[didx_v.at[i]],
                                  sem.at[r]).wait()
            return 0

        lax.fori_loop(n_ch - 8, n_ch, drain, 0)
        plsc.subcore_barrier()

        @pl.when(sid < n_z)
        def _():
            pltpu.sync_copy(acc_sh.at[pl.ds(sid * ZCH, ZCH)], zbuf_v)
            pltpu.sync_copy(zbuf_v, out_hbm.at[pl.ds(cid * N + sid * ZCH, ZCH)])

    return deg_k


@functools.lru_cache(maxsize=None)
def _make_agg_kernel(N, E, F):
    e_per_w = E // _NW
    CH = 40
    NBUF = 5
    GRP = 10
    n_ch = e_per_w // CH
    n_grp = n_ch // GRP
    assert n_ch % NBUF == 0 and n_ch % GRP == 0 and GRP % NBUF == 0
    n_rz = N // CH
    n_rounds = (n_rz + _NS - 1) // _NS
    mesh = plsc.VectorSubcoreMesh(core_axis_name="c", subcore_axis_name="s")

    @functools.partial(
        pl.kernel,
        mesh=mesh,
        out_type=jax.ShapeDtypeStruct((_NC, N, F), jnp.float32),
        scratch_types=[
            pltpu.VMEM((e_per_w,), jnp.int32),
            pltpu.VMEM((2, GRP, CH), jnp.int32),
            pltpu.VMEM((NBUF, CH, F), jnp.float32),
            pltpu.VMEM_SHARED((N, F), jnp.float32),
            pltpu.SemaphoreType.DMA((NBUF,)),
            pltpu.SemaphoreType.DMA((2,)),
        ],
    )
    def agg_k(hs_hbm, eflat_hbm, eidx_hbm, out_hbm, sidx_v, dring_v, rows_v,
              acc_sh, sem, semd):
        cid = lax.axis_index("c")
        sid = lax.axis_index("s")
        wid = sid * _NC + cid

        pltpu.async_copy(eflat_hbm.at[pl.ds(wid * e_per_w, e_per_w)],
                         sidx_v, sem.at[0])

        def fill_zeros(i, _):
            for j in range(F // 16):
                rows_v[0, i, pl.ds(j * 16, 16)] = jnp.zeros((16,),
                                                            jnp.float32)
            return 0

        lax.fori_loop(0, CH, fill_zeros, 0)

        def zloop(k, _):
            c = sid + k * _NS

            @pl.when(c < n_rz)
            def _():
                pltpu.sync_copy(rows_v.at[0], acc_sh.at[pl.ds(c * CH, CH)])

            return 0

        lax.fori_loop(0, n_rounds, zloop, 0)
        pltpu.make_async_copy(eflat_hbm.at[pl.ds(wid * e_per_w, e_per_w)],
                              sidx_v, sem.at[0]).wait()
        plsc.subcore_barrier()

        pltpu.async_copy(eidx_hbm.at[1, wid, 0], dring_v.at[0], semd.at[0])
        for b in range(NBUF):
            pltpu.async_copy(hs_hbm.at[sidx_v.at[pl.ds(b * CH, CH)]],
                             rows_v.at[b], sem.at[b])
        pltpu.make_async_copy(eidx_hbm.at[1, wid, 0], dring_v.at[0],
                              semd.at[0]).wait()

        def gbody(g, _):
            r = lax.rem(g, 2)

            @pl.when(g > 0)
            def _():
                pltpu.make_async_copy(eidx_hbm.at[1, wid, g],
                                      dring_v.at[r], semd.at[r]).wait()

            @pl.when(g + 1 < n_grp)
            def _():
                pltpu.async_copy(eidx_hbm.at[1, wid, g + 1],
                                 dring_v.at[1 - r], semd.at[1 - r])

            for k in range(GRP):
                b = k % NBUF
                i = g * GRP + k
                pltpu.make_async_copy(
                    hs_hbm.at[sidx_v.at[pl.ds(i * CH, CH)]],
                    rows_v.at[b], sem.at[b]
                ).wait()
                pltpu.sync_copy(rows_v.at[b], acc_sh.at[dring_v.at[r, k]],
                                add=True)
                j = i + NBUF

                @pl.when(j < n_ch)
                def _():
                    pltpu.async_copy(
                        hs_hbm.at[sidx_v.at[pl.ds(j * CH, CH)]],
                        rows_v.at[b], sem.at[b])

            return 0

        lax.fori_loop(0, n_grp, gbody, 0)
        plsc.subcore_barrier()

        def cpout(k, _):
            c = sid + k * _NS
            r = lax.rem(k, 2)

            @pl.when(c < n_rz)
            def _():
                @pl.when(k >= 2)
                def _():
                    pltpu.make_async_copy(
                        rows_v.at[r],
                        out_hbm.at[cid, pl.ds((c - 2 * _NS) * CH, CH)],
                        sem.at[r]).wait()

                pltpu.sync_copy(acc_sh.at[pl.ds(c * CH, CH)], rows_v.at[r])
                pltpu.async_copy(rows_v.at[r],
                                 out_hbm.at[cid, pl.ds(c * CH, CH)],
                                 sem.at[r])

            return 0

        lax.fori_loop(0, n_rounds, cpout, 0)

        def cdrain(k, _):
            c = sid + k * _NS
            r = lax.rem(k, 2)

            @pl.when(c < n_rz)
            def _():
                pltpu.make_async_copy(rows_v.at[r],
                                      out_hbm.at[cid, pl.ds(c * CH, CH)],
                                      sem.at[r]).wait()

            return 0

        lax.fori_loop(n_rounds - 2, n_rounds, cdrain, 0)

    return agg_k


_BLK = 2000


def _tc_first(degT, x, W1):
    N, IN = x.shape
    H = W1.shape[1]

    def body(deg_ref, x_ref, w_ref, hs_ref, dinv_ref):
        d = deg_ref[:, 0] + deg_ref[:, 1] + 1.0
        dinv = lax.rsqrt(d)
        h = jnp.dot(x_ref[...], w_ref[...], preferred_element_type=jnp.float32)
        hs_ref[...] = h * dinv[:, None]
        dinv_ref[...] = dinv[:, None]

    return pl.pallas_call(
        body,
        grid=(N // _BLK,),
        in_specs=[
            pl.BlockSpec((_BLK, 2), lambda i: (i, 0)),
            pl.BlockSpec((_BLK, IN), lambda i: (i, 0)),
            pl.BlockSpec((IN, H), lambda i: (0, 0)),
        ],
        out_specs=[
            pl.BlockSpec((_BLK, H), lambda i: (i, 0)),
            pl.BlockSpec((_BLK, 1), lambda i: (i, 0)),
        ],
        out_shape=[
            jax.ShapeDtypeStruct((N, H), jnp.float32),
            jax.ShapeDtypeStruct((N, 1), jnp.float32),
        ],
    )(degT, x, W1)


def _tc_mid(dinv, hs1, agg1, W2, b1):
    N, H = hs1.shape

    def body(dinv_ref, hs_ref, agg_ref, w_ref, b_ref, out_ref):
        dinv = dinv_ref[...]
        a = agg_ref[0] + agg_ref[1] + hs_ref[...]
        t = jnp.maximum(a * dinv + b_ref[...], 0.0)
        out_ref[...] = (
            jnp.dot(t, w_ref[...], preferred_element_type=jnp.float32) * dinv
        )

    return pl.pallas_call(
        body,
        grid=(N // _BLK,),
        in_specs=[
            pl.BlockSpec((_BLK, 1), lambda i: (i, 0)),
            pl.BlockSpec((_BLK, H), lambda i: (i, 0)),
            pl.BlockSpec((2, _BLK, H), lambda i: (0, i, 0)),
            pl.BlockSpec((H, H), lambda i: (0, 0)),
            pl.BlockSpec((1, H), lambda i: (0, 0)),
        ],
        out_specs=pl.BlockSpec((_BLK, H), lambda i: (i, 0)),
        out_shape=jax.ShapeDtypeStruct((N, H), jnp.float32),
    )(dinv, hs1, agg1, W2, b1)


def _tc_last(dinv, hs2, agg2, b2, batchT, Wfc, bfc, G):
    N, H = hs2.shape
    OUT = Wfc.shape[1]
    nsteps = N // _BLK

    def body(dinv_ref, hs_ref, agg_ref, b_ref, batch_ref, wfc_ref, bfc_ref,
             y_ref, u_ref):
        i = pl.program_id(0)
        dinv = dinv_ref[...]
        z = (agg_ref[0] + agg_ref[1] + hs_ref[...]) * dinv + b_ref[...]
        cols = lax.broadcasted_iota(jnp.int32, (_BLK, G), 1)
        oh = (batch_ref[...] == cols).astype(jnp.float32)
        pu = lax.dot_general(
            oh, z, (((0,), (0,)), ((), ())),
            preferred_element_type=jnp.float32,
        )

        @pl.when(i == 0)
        def _():
            u_ref[...] = jnp.zeros_like(u_ref)

        u_ref[...] += pu

        @pl.when(i == nsteps - 1)
        def _():
            y_ref[...] = (
                jnp.dot(u_ref[...], wfc_ref[...],
                        preferred_element_type=jnp.float32)
                + bfc_ref[...]
            )

    return pl.pallas_call(
        body,
        grid=(nsteps,),
        in_specs=[
            pl.BlockSpec((_BLK, 1), lambda i: (i, 0)),
            pl.BlockSpec((_BLK, H), lambda i: (i, 0)),
            pl.BlockSpec((2, _BLK, H), lambda i: (0, i, 0)),
            pl.BlockSpec((1, H), lambda i: (0, 0)),
            pl.BlockSpec((_BLK, 1), lambda i: (i, 0)),
            pl.BlockSpec((H, OUT), lambda i: (0, 0)),
            pl.BlockSpec((1, OUT), lambda i: (0, 0)),
        ],
        out_specs=[
            pl.BlockSpec((G, OUT), lambda i: (0, 0)),
            pl.BlockSpec((G, H), lambda i: (0, 0)),
        ],
        out_shape=[
            jax.ShapeDtypeStruct((G, OUT), jnp.float32),
            jax.ShapeDtypeStruct((G, H), jnp.float32),
        ],
    )(dinv, hs2, agg2, b2, batchT, Wfc, bfc)


def kernel(x, edge_index, batch, W1, b1, W2, b2, Wfc, bfc):
    N, IN = x.shape
    H = W1.shape[1]
    OUT = Wfc.shape[1]
    E = edge_index.shape[1]
    G = 64

    eflat = edge_index.reshape(2 * E)
    eidx4 = edge_index.reshape(2, _NW, -1, 80)
    eidx5 = edge_index.reshape(2, _NW, -1, 10, 40)

    degp = _make_deg_kernel(N, E)(eidx4)
    degT = jnp.transpose(degp.reshape(_NC, N))

    hs1, dinv = _tc_first(degT, x, W1)
    agg1 = _make_agg_kernel(N, E, H)(hs1, eflat, eidx5)
    hs2 = _tc_mid(dinv, hs1, agg1, W2, b1.reshape(1, H))
    agg2 = _make_agg_kernel(N, E, H)(hs2, eflat, eidx5)
    y, u = _tc_last(dinv, hs2, agg2, b2.reshape(1, H),
                    batch.reshape(N, 1), Wfc, bfc.reshape(1, OUT), G)
    return (y, u)
